# baseline (speedup 1.0000x reference)
import dataclasses

import jax
import jax.numpy as jnp
from jax import lax
from jax.experimental import pallas as pl
from jax.experimental.pallas import tpu as pltpu
from jax.experimental.pallas import tpu_sc as plsc

N = 100000
H = 16
NP = 100352
E = 1600000
WSZ = 256
NWIN = 396
EPT = WSZ * NWIN
EPAD = EPT * 16
CHUNK = NP // 16
EW = 112
NLAYERS = 3


def _rsqrt16(d):
    i = plsc.bitcast(d, jnp.int32)
    i = jnp.int32(0x5F3759DF) - lax.shift_right_logical(i, 1)
    y = plsc.bitcast(i, jnp.float32)
    for _ in range(3):
        y = y * (1.5 - 0.5 * d * y * y)
    return jnp.where(d > 0.0, y, 0.0)


def _bcast16(ref, i):
    return plsc.load_gather(ref, [jnp.full((16,), i, jnp.int32)])


def _sc_body(emb_f, rc3, out,
             y_f, sum_f,
             accum, dd,
             zb, ones,
             ab0, dw0, sb0,
             ic0, ic1, ic2, ic3, ic4, ic5,
             v0, v1, v2,
             semV0, semV1, semV2, semS0, semS1, semS2,
             semI0, semI1, semI2, semI3, semI4, semI5):
    c = lax.axis_index("c")
    t = lax.axis_index("s")
    r0 = t * CHUNK
    i0 = t * (NWIN * 2)
    hoff = c * NP

    ics = (ic0, ic1, ic2, ic3, ic4, ic5)
    semIs = (semI0, semI1, semI2, semI3, semI4, semI5)
    vs = (v0, v1, v2)
    semVs = (semV0, semV1, semV2)
    semSs = (semS0, semS1, semS2)

    z16 = jnp.zeros((16,), jnp.float32)

    @pl.loop(0, EW)
    def _(i):
        zb[i, :] = z16

    @pl.loop(0, EW, step=16)
    def _(i):
        dw0[pl.ds(i, 16)] = z16

    @pl.loop(0, WSZ, step=16)
    def _(i):
        ones[pl.ds(i, 16)] = jnp.ones((16,), jnp.float32)

    @pl.loop(0, CHUNK, step=EW)
    def _(w):
        pltpu.sync_copy(zb, accum.at[pl.ds(r0 + w, EW), :])
        pltpu.sync_copy(dw0, dd.at[pl.ds(r0 + w, EW)])

    plsc.subcore_barrier()

    def _idx_fire(w, b):
        pltpu.async_copy(rc3.at[pl.ds(i0 + w * 2, 2), :], ics[b],
                         semIs[b])

    def _idx_wait(b):
        pltpu.make_async_copy(rc3.at[pl.ds(0, 2), :], ics[b],
                              semIs[b]).wait()

    _idx_fire(0, 0)

    @pl.loop(0, NWIN, step=2)
    def _(w):
        _idx_fire(w + 1, 1)
        _idx_wait(0)
        pltpu.sync_copy(ones, dd.at[ic0.at[1]], add=True)

        @pl.when(w + 2 < NWIN)
        def _():
            _idx_fire(w + 2, 0)

        _idx_wait(1)
        pltpu.sync_copy(ones, dd.at[ic1.at[1]], add=True)

    plsc.subcore_barrier()

    @pl.loop(0, CHUNK, step=EW)
    def _(w):
        g0 = r0 + w
        pltpu.sync_copy(dd.at[pl.ds(g0, EW)], dw0)

        @pl.loop(0, EW, step=16)
        def _(i):
            dw0[pl.ds(i, 16)] = _rsqrt16(dw0[pl.ds(i, 16)])

        pltpu.sync_copy(dw0, dd.at[pl.ds(g0, EW)])
        pltpu.sync_copy(emb_f.at[pl.ds(hoff + g0, EW), :], ab0)

        @pl.loop(0, EW)
        def _(i):
            ab0[i, :] = ab0[i, :] * _bcast16(dw0, i)

        pltpu.sync_copy(ab0, y_f.at[pl.ds(hoff + g0, EW), :])

    plsc.subcore_barrier()

    def _gather_fire(b, p):
        @pl.loop(0, WSZ, step=16)
        def _(i):
            ics[b][0, pl.ds(i, 16)] = ics[b][0, pl.ds(i, 16)] + hoff
        pltpu.async_copy(y_f.at[ics[b].at[0]], vs[p], semVs[p])

    def _gather_wait(b, p):
        pltpu.make_async_copy(y_f.at[ics[b].at[0]], vs[p],
                              semVs[p]).wait()

    def _scat_wait(p):
        pltpu.make_async_copy(y_f.at[pl.ds(0, WSZ), :], vs[p],
                              semSs[p]).wait()

    for l in range(NLAYERS):
        for b in range(4):
            _idx_fire(b, b)
        _idx_wait(0)
        _gather_fire(0, 0)
        _idx_wait(1)
        _gather_fire(1, 1)

        @pl.loop(0, NWIN, step=6)
        def _(w):
            for u in range(6):
                wu = w + u
                b3 = u % 3
                _gather_wait(u, b3)
                pltpu.async_copy(vs[b3], accum.at[ics[u].at[1]],
                                 semSs[b3], add=True)

                @pl.when((wu + 2 < NWIN) & (wu >= 1))
                def _():
                    _scat_wait((u + 2) % 3)

                @pl.when(wu + 2 < NWIN)
                def _():
                    _idx_wait((u + 2) % 6)
                    _gather_fire((u + 2) % 6, (u + 2) % 3)

                @pl.when(wu + 4 < NWIN)
                def _():
                    _idx_fire(wu + 4, (u + 4) % 6)

        for b in range(3):
            _scat_wait(b)
        plsc.subcore_barrier()

        @pl.loop(0, CHUNK, step=EW)
        def _(w):
            g0 = r0 + w
            pltpu.sync_copy(accum.at[pl.ds(g0, EW), :], ab0)
            pltpu.sync_copy(zb, accum.at[pl.ds(g0, EW), :])
            pltpu.sync_copy(dd.at[pl.ds(g0, EW)], dw0)
            if l == 0:
                pltpu.sync_copy(emb_f.at[pl.ds(hoff + g0, EW), :], sb0)
            else:
                pltpu.sync_copy(sum_f.at[pl.ds(hoff + g0, EW), :], sb0)

            if l < NLAYERS - 1:
                @pl.loop(0, EW)
                def _(i):
                    d = _bcast16(dw0, i)
                    x = ab0[i, :] * d
                    sb0[i, :] = sb0[i, :] + x
                    ab0[i, :] = x * d
                pltpu.sync_copy(sb0, sum_f.at[pl.ds(hoff + g0, EW), :])
                pltpu.sync_copy(ab0, y_f.at[pl.ds(hoff + g0, EW), :])
            else:
                @pl.loop(0, EW)
                def _(i):
                    x = ab0[i, :] * _bcast16(dw0, i)
                    sb0[i, :] = (sb0[i, :] + x) * 0.25
                pltpu.sync_copy(
                    sb0, out.at[pl.ds(g0, EW), pl.ds(c * H, H)])

        plsc.subcore_barrier()


@jax.jit
def _lightgcn_sc(emb_f, rc3):
    cp = pltpu.CompilerParams(use_tc_tiling_on_sc=False)
    if "needs_layout_passes" in pltpu.CompilerParams.__dataclass_fields__:
        cp = dataclasses.replace(cp, needs_layout_passes=False)
    mesh = plsc.VectorSubcoreMesh(core_axis_name="c", subcore_axis_name="s")
    k = pl.kernel(
        _sc_body,
        out_type=jax.ShapeDtypeStruct((NP, 2 * H), jnp.float32),
        mesh=mesh,
        scratch_types=[
            pltpu.HBM((2 * NP, H), jnp.float32),
            pltpu.HBM((2 * NP, H), jnp.float32),
            pltpu.VMEM_SHARED((NP, H), jnp.float32),
            pltpu.VMEM_SHARED((NP,), jnp.float32),
            pltpu.VMEM((EW, H), jnp.float32),
            pltpu.VMEM((WSZ,), jnp.float32),
            pltpu.VMEM((EW, H), jnp.float32),
            pltpu.VMEM((EW,), jnp.float32),
            pltpu.VMEM((EW, H), jnp.float32),
            pltpu.VMEM((2, WSZ), jnp.int32),
            pltpu.VMEM((2, WSZ), jnp.int32),
            pltpu.VMEM((2, WSZ), jnp.int32),
            pltpu.VMEM((2, WSZ), jnp.int32),
            pltpu.VMEM((2, WSZ), jnp.int32),
            pltpu.VMEM((2, WSZ), jnp.int32),
            pltpu.VMEM((WSZ, H), jnp.float32),
            pltpu.VMEM((WSZ, H), jnp.float32),
            pltpu.VMEM((WSZ, H), jnp.float32),
            pltpu.SemaphoreType.DMA,
            pltpu.SemaphoreType.DMA,
            pltpu.SemaphoreType.DMA,
            pltpu.SemaphoreType.DMA,
            pltpu.SemaphoreType.DMA,
            pltpu.SemaphoreType.DMA,
            pltpu.SemaphoreType.DMA,
            pltpu.SemaphoreType.DMA,
            pltpu.SemaphoreType.DMA,
            pltpu.SemaphoreType.DMA,
            pltpu.SemaphoreType.DMA,
            pltpu.SemaphoreType.DMA,
        ],
        compiler_params=cp,
    )
    return k(emb_f, rc3)


def kernel(emb, edge_index):
    emb = emb.astype(jnp.float32)
    row = edge_index[0].astype(jnp.int32)
    col = edge_index[1].astype(jnp.int32)
    npad = EPAD - E
    ar = jnp.arange(npad, dtype=jnp.int32)
    pad_idx = N + ar % (NP - N)
    row_w = jnp.concatenate([row, pad_idx]).reshape(16, NWIN, 1, WSZ)
    col_w = jnp.concatenate([col, pad_idx]).reshape(16, NWIN, 1, WSZ)
    rc3 = jnp.concatenate([col_w, row_w], axis=2).reshape(-1, WSZ)
    emb_f = jnp.zeros((2 * NP, H), jnp.float32)
    emb_f = emb_f.at[:N].set(emb[:, :H]).at[NP:NP + N].set(emb[:, H:])
    final = _lightgcn_sc(emb_f, rc3)
    return final[:40000], final[40000:90000], final[90000:N]
